# baseline (speedup 1.0000x reference)
import jax
import jax.numpy as jnp
from jax import lax
from jax.experimental import pallas as pl
from jax.experimental.pallas import tpu as pltpu
from jax.experimental.pallas import tpu_sc as plsc

N_NUM = 13
N_CAT = 26
D = 32
BATCH = 4096
CARD = 100000
NTOK = N_NUM + N_CAT

NC = 2
NS = 16
NW = NC * NS
ROWS_W = BATCH // NW
BB = 16
NBLK = ROWS_W // BB


def _body(x_hbm, wnum_hbm, bnum_hbm, table_hbm, bcat_hbm, out_hbm,
          x_v, wnum_v, bnum_v, bcat_v, idx_v,
          tb0, tb1, sem_g, sem_o0, sem_o1):
    cid = lax.axis_index("c")
    sid = lax.axis_index("s")
    wid = sid * NC + cid
    base = wid * ROWS_W

    pltpu.sync_copy(x_hbm.at[pl.ds(base, ROWS_W)], x_v)
    pltpu.sync_copy(wnum_hbm, wnum_v)
    pltpu.sync_copy(bnum_hbm, bnum_v)
    pltpu.sync_copy(bcat_hbm, bcat_v)

    lane = jnp.arange(16, dtype=jnp.int32)
    offs1 = lane * CARD
    offs2 = (lane + 10) * CARD

    tokbufs = (tb0, tb1)
    sems_o = (sem_o0, sem_o1)
    out_handles = [None] * NBLK

    for blk in range(NBLK):
        tb = tokbufs[blk & 1]
        sem_o = sems_o[blk & 1]

        if blk >= 2:
            out_handles[blk - 2].wait()

        @pl.loop(0, BB)
        def _build(b2, _blk=blk):
            row = _blk * BB + b2
            c1 = x_v[row, pl.ds(13, 16)] + offs1
            c2 = x_v[row, pl.ds(23, 16)] + offs2
            idx_v[b2, pl.ds(0, 16)] = c1
            idx_v[b2, pl.ds(10, 16)] = c2

        gh = []
        for b2 in range(BB):
            gh.append(pltpu.async_copy(
                table_hbm.at[idx_v.at[b2]],
                tb.at[b2, pl.ds(N_NUM, N_CAT), :], sem_g))

        @pl.loop(0, BB)
        def _numeric(b2, _blk=blk):
            row = _blk * BB + b2
            xrow = x_v[row, pl.ds(0, 16)].astype(jnp.float32)
            dnums = lax.GatherDimensionNumbers(
                offset_dims=(), collapsed_slice_dims=(0,),
                start_index_map=(0,))
            for i in range(N_NUM):
                iv = jnp.full((16, 1), i, dtype=jnp.int32)
                xf = lax.gather(
                    xrow, iv, dnums, slice_sizes=(1,),
                    mode=lax.GatherScatterMode.PROMISE_IN_BOUNDS)
                for h in range(2):
                    s = pl.ds(h * 16, 16)
                    tb[b2, i, s] = xf * wnum_v[i, s] + bnum_v[i, s]

        for h in gh:
            h.wait()

        @pl.loop(0, BB)
        def _bias(b2):
            for j in range(N_CAT):
                for h in range(2):
                    s = pl.ds(h * 16, 16)
                    tb[b2, N_NUM + j, s] = tb[b2, N_NUM + j, s] + bcat_v[j, s]

        bstart = base + blk * BB
        out_handles[blk] = pltpu.async_copy(
            tb, out_hbm.at[pl.ds(bstart, BB), :, :], sem_o)

    for blk in range(NBLK - 2, NBLK):
        out_handles[blk].wait()


@jax.jit
def _tokenizer(x, W_num, b_num, table, b_cat):
    mesh = plsc.VectorSubcoreMesh(core_axis_name="c", subcore_axis_name="s",
                                  num_cores=NC, num_subcores=NS)
    f = pl.kernel(
        _body,
        out_type=jax.ShapeDtypeStruct((BATCH, NTOK, D), jnp.float32),
        mesh=mesh,
        scratch_types=[
            pltpu.VMEM((ROWS_W, NTOK), jnp.int32),
            pltpu.VMEM((N_NUM, D), jnp.float32),
            pltpu.VMEM((N_NUM, D), jnp.float32),
            pltpu.VMEM((N_CAT, D), jnp.float32),
            pltpu.VMEM((BB, N_CAT), jnp.int32),
            pltpu.VMEM((BB, NTOK, D), jnp.float32),
            pltpu.VMEM((BB, NTOK, D), jnp.float32),
            pltpu.SemaphoreType.DMA,
            pltpu.SemaphoreType.DMA,
            pltpu.SemaphoreType.DMA,
        ],
        compiler_params=pltpu.CompilerParams(use_tc_tiling_on_sc=False),
        name="rtdl_tokenizer_sc",
    )
    return f(x, W_num, b_num, table, b_cat)


def kernel(x, W_num, b_num, table, b_cat):
    return _tokenizer(x, W_num, b_num, table, b_cat)
